# Initial kernel scaffold; baseline (speedup 1.0000x reference)
import jax
import jax.numpy as jnp
from jax.experimental import pallas as pl


def kernel(x, edge_index, Ws1, Wn1, b1, Ws2, Wn2, b2, Ws3, Wn3, b3, Ws4, Wn4, b4):
    raise NotImplementedError("write your pallas kernel here")


# speedup vs baseline: 3.2561x; 3.2561x over previous
import functools

import jax
import jax.numpy as jnp
from jax import lax
from jax.experimental import pallas as pl
from jax.experimental.pallas import tpu as pltpu
from jax.experimental.pallas import tpu_sc as plsc

N = 10000
N_PAD = 10240
E = 320000
NCORES = 2
NTILES = 16
ROWS_PER_TILE = N_PAD // NTILES
B = 128
CHUNK = 32
BPT = 160
EPT = BPT * B
E_PAD = EPT * NTILES


def _agg_kernel(Fc):
    mesh = plsc.VectorSubcoreMesh(core_axis_name="c", subcore_axis_name="s")

    @functools.partial(
        pl.kernel,
        out_type=jax.ShapeDtypeStruct((NCORES * N_PAD, Fc), jnp.float32),
        mesh=mesh,
        scratch_types=[
            pltpu.VMEM((CHUNK * B,), jnp.int32),
            pltpu.VMEM((CHUNK * B,), jnp.int32),
            pltpu.VMEM((B,), jnp.int32),
            pltpu.VMEM((B,), jnp.int32),
            pltpu.VMEM((B, Fc), jnp.float32),
            pltpu.VMEM_SHARED((N_PAD, Fc), jnp.float32),
        ],
        compiler_params=pltpu.CompilerParams(use_tc_tiling_on_sc=False),
    )
    def agg(h_hbm, src_hbm, dst_hbm, out_hbm, src_ch, dst_ch, src_b, dst_b,
            rows, acc):
        c = lax.axis_index("c")
        s = lax.axis_index("s")
        zero16 = jnp.zeros((16,), jnp.float32)

        def zrow(i, _):
            for k in range(Fc // 16):
                rows[i, pl.ds(k * 16, 16)] = zero16
            return 0
        lax.fori_loop(0, B, zrow, 0)
        row0 = s * ROWS_PER_TILE
        for k in range(ROWS_PER_TILE // B):
            pltpu.sync_copy(rows, acc.at[pl.ds(row0 + k * B, B)])
        plsc.subcore_barrier()

        ebase = s * EPT
        coff = c * N_PAD

        def chunk_body(o, _):
            cbase = ebase + o * (CHUNK * B)
            pltpu.sync_copy(src_hbm.at[pl.ds(cbase, CHUNK * B)], src_ch)
            pltpu.sync_copy(dst_hbm.at[pl.ds(cbase, CHUNK * B)], dst_ch)

            def body(i, _):
                base = i * B
                for k in range(B // 16):
                    src_b[pl.ds(k * 16, 16)] = src_ch[pl.ds(base + k * 16, 16)] + coff
                    dst_b[pl.ds(k * 16, 16)] = dst_ch[pl.ds(base + k * 16, 16)]
                pltpu.sync_copy(h_hbm.at[src_b], rows)
                pltpu.sync_copy(rows, acc.at[dst_b], add=True)
                return 0
            lax.fori_loop(0, CHUNK, body, 0)
            return 0
        lax.fori_loop(0, BPT // CHUNK, chunk_body, 0)
        plsc.subcore_barrier()

        obase = c * N_PAD + row0
        for k in range(ROWS_PER_TILE // B):
            pltpu.sync_copy(acc.at[pl.ds(row0 + k * B, B)], rows)
            pltpu.sync_copy(rows, out_hbm.at[pl.ds(obase + k * B, B)])

    return agg


def _deg_kernel():
    Fc = 16
    mesh = plsc.VectorSubcoreMesh(core_axis_name="c", subcore_axis_name="s")

    @functools.partial(
        pl.kernel,
        out_type=jax.ShapeDtypeStruct((N_PAD, Fc), jnp.float32),
        mesh=mesh,
        scratch_types=[
            pltpu.VMEM((CHUNK * B,), jnp.int32),
            pltpu.VMEM((B,), jnp.int32),
            pltpu.VMEM((B, Fc), jnp.float32),
            pltpu.VMEM_SHARED((N_PAD, Fc), jnp.float32),
        ],
        compiler_params=pltpu.CompilerParams(use_tc_tiling_on_sc=False),
    )
    def deg(dst_hbm, out_hbm, dst_ch, dst_b, rows, acc):
        c = lax.axis_index("c")
        s = lax.axis_index("s")
        zero16 = jnp.zeros((16,), jnp.float32)
        one16 = jnp.ones((16,), jnp.float32)

        def zrow(i, _):
            rows[i, pl.ds(0, 16)] = zero16
            return 0
        lax.fori_loop(0, B, zrow, 0)
        row0 = s * ROWS_PER_TILE
        for k in range(ROWS_PER_TILE // B):
            pltpu.sync_copy(rows, acc.at[pl.ds(row0 + k * B, B)])
        plsc.subcore_barrier()

        def orow(i, _):
            rows[i, pl.ds(0, 16)] = one16
            return 0
        lax.fori_loop(0, B, orow, 0)

        ebase = s * EPT

        def chunk_body(o, _):
            cbase = ebase + o * (CHUNK * B)
            pltpu.sync_copy(dst_hbm.at[pl.ds(cbase, CHUNK * B)], dst_ch)

            def body(i, _):
                base = i * B
                for k in range(B // 16):
                    dst_b[pl.ds(k * 16, 16)] = dst_ch[pl.ds(base + k * 16, 16)]
                pltpu.sync_copy(rows, acc.at[dst_b], add=True)
                return 0
            lax.fori_loop(0, CHUNK, body, 0)
            return 0
        lax.fori_loop(0, BPT // CHUNK, chunk_body, 0)
        plsc.subcore_barrier()

        @pl.when(c == 0)
        def _():
            for k in range(ROWS_PER_TILE // B):
                pltpu.sync_copy(acc.at[pl.ds(row0 + k * B, B)], rows)
                pltpu.sync_copy(rows, out_hbm.at[pl.ds(row0 + k * B, B)])

    return deg


BM = 512
NB = N_PAD // BM


def _tc_layer(Hc, Fo, relu):
    Co = 2
    Fco = Fo // Co

    def body(h_ref, agg_ref, deg_ref, ws_ref, wn_ref, b_ref, out_ref):
        scale = 1.0 / jnp.maximum(deg_ref[...], 1.0)
        acc = jnp.dot(h_ref[0], ws_ref[:Hc], preferred_element_type=jnp.float32)
        acc += jnp.dot(h_ref[1], ws_ref[Hc:], preferred_element_type=jnp.float32)
        acc += jnp.dot(agg_ref[0] * scale, wn_ref[:Hc], preferred_element_type=jnp.float32)
        acc += jnp.dot(agg_ref[1] * scale, wn_ref[Hc:], preferred_element_type=jnp.float32)
        acc += b_ref[...]
        if relu:
            acc = jnp.maximum(acc, 0.0)
        out_ref[0] = acc

    return pl.pallas_call(
        body,
        grid=(Co, NB),
        in_specs=[
            pl.BlockSpec((2, BM, Hc), lambda co, nb: (0, nb, 0)),
            pl.BlockSpec((2, BM, Hc), lambda co, nb: (0, nb, 0)),
            pl.BlockSpec((BM, 1), lambda co, nb: (nb, 0)),
            pl.BlockSpec((2 * Hc, Fco), lambda co, nb: (0, co)),
            pl.BlockSpec((2 * Hc, Fco), lambda co, nb: (0, co)),
            pl.BlockSpec((1, Fco), lambda co, nb: (0, co)),
        ],
        out_specs=pl.BlockSpec((1, BM, Fco), lambda co, nb: (co, nb, 0)),
        out_shape=jax.ShapeDtypeStruct((Co, N_PAD, Fco), jnp.float32),
    )


def _tc_pre4():
    Hc, Fco = 128, 32

    def body(h_ref, wn_ref, out_ref):
        acc = jnp.dot(h_ref[0], wn_ref[:Hc], preferred_element_type=jnp.float32)
        acc += jnp.dot(h_ref[1], wn_ref[Hc:], preferred_element_type=jnp.float32)
        out_ref[0] = acc[:, :Fco]
        out_ref[1] = acc[:, Fco:]

    return pl.pallas_call(
        body,
        grid=(NB,),
        in_specs=[
            pl.BlockSpec((2, BM, Hc), lambda nb: (0, nb, 0)),
            pl.BlockSpec((2 * Hc, 2 * Fco), lambda nb: (0, 0)),
        ],
        out_specs=pl.BlockSpec((2, BM, Fco), lambda nb: (0, nb, 0)),
        out_shape=jax.ShapeDtypeStruct((2, N_PAD, Fco), jnp.float32),
    )


def _tc_final():
    Hc = 128

    def body(h_ref, agg_ref, deg_ref, ws_ref, b_ref, out_ref):
        scale = 1.0 / jnp.maximum(deg_ref[...], 1.0)
        acc = jnp.dot(h_ref[0], ws_ref[:Hc], preferred_element_type=jnp.float32)
        acc += jnp.dot(h_ref[1], ws_ref[Hc:], preferred_element_type=jnp.float32)
        acc += jnp.concatenate([agg_ref[0], agg_ref[1]], axis=1) * scale
        acc += b_ref[...]
        out_ref[...] = acc

    return pl.pallas_call(
        body,
        grid=(NB,),
        in_specs=[
            pl.BlockSpec((2, BM, Hc), lambda nb: (0, nb, 0)),
            pl.BlockSpec((2, BM, 32), lambda nb: (0, nb, 0)),
            pl.BlockSpec((BM, 1), lambda nb: (nb, 0)),
            pl.BlockSpec((256, 64), lambda nb: (0, 0)),
            pl.BlockSpec((1, 64), lambda nb: (0, 0)),
        ],
        out_specs=pl.BlockSpec((BM, 64), lambda nb: (nb, 0)),
        out_shape=jax.ShapeDtypeStruct((N_PAD, 64), jnp.float32),
    )


_agg64 = _agg_kernel(64)
_agg128 = _agg_kernel(128)
_agg32 = _agg_kernel(32)
_deg = _deg_kernel()
_layer1 = _tc_layer(64, 256, True)
_layer23 = _tc_layer(128, 256, True)
_pre4 = _tc_pre4()
_final4 = _tc_final()


def kernel(x, edge_index, Ws1, Wn1, b1, Ws2, Wn2, b2, Ws3, Wn3, b3,
           Ws4, Wn4, b4):
    src = edge_index[0]
    dst = edge_index[1]
    src_p = jnp.pad(src, (0, E_PAD - E))
    dst_p = jnp.pad(dst, (0, E_PAD - E), constant_values=N_PAD - 1)

    x_p = jnp.pad(x, ((0, N_PAD - N), (0, 0)))
    x_c = x_p.reshape(N_PAD, 2, 64).transpose(1, 0, 2)

    deg = _deg(dst_p)[:, 0:1]

    agg1 = _agg64(x_c.reshape(2 * N_PAD, 64), src_p, dst_p).reshape(2, N_PAD, 64)
    h1 = _layer1(x_c, agg1, deg, Ws1, Wn1, b1.reshape(1, -1))

    agg2 = _agg128(h1.reshape(2 * N_PAD, 128), src_p, dst_p).reshape(2, N_PAD, 128)
    h2 = _layer23(h1, agg2, deg, Ws2, Wn2, b2.reshape(1, -1))

    agg3 = _agg128(h2.reshape(2 * N_PAD, 128), src_p, dst_p).reshape(2, N_PAD, 128)
    h3 = _layer23(h2, agg3, deg, Ws3, Wn3, b3.reshape(1, -1))

    hn4 = _pre4(h3, Wn4)
    agg4 = _agg32(hn4.reshape(2 * N_PAD, 32), src_p, dst_p).reshape(2, N_PAD, 32)
    out = _final4(h3, agg4, deg, Ws4, b4.reshape(1, -1))
    return out[:N]
